# baseline (device time: 58003 ns/iter reference)
import jax
import jax.numpy as jnp
from jax import lax
from jax.experimental import pallas as pl
from jax.experimental.pallas import tpu as pltpu

N_DEV = 4
N_GLOBAL = 8192
EPS = 1e-5
BLOCK_M = 512


def _partial_body(x_ref, out_ref):
    x = x_ref[...]
    out_ref[...] = jnp.sum(x * x, axis=1, keepdims=True)


def _allreduce_body(p_ref, out_ref, comm_ref, send_sems, recv_sems):
    me = lax.axis_index("i")

    barrier = pltpu.get_barrier_semaphore()
    for k in range(1, N_DEV):
        peer = (me + k) % N_DEV
        pl.semaphore_signal(
            barrier, inc=1,
            device_id=(peer,), device_id_type=pl.DeviceIdType.MESH,
        )
    pl.semaphore_wait(barrier, N_DEV - 1)

    comm_ref[me] = p_ref[...]

    sends = []
    for k in range(1, N_DEV):
        peer = (me + k) % N_DEV
        rdma = pltpu.make_async_remote_copy(
            src_ref=p_ref,
            dst_ref=comm_ref.at[me],
            send_sem=send_sems.at[k - 1],
            recv_sem=recv_sems.at[me],
            device_id=(peer,),
            device_id_type=pl.DeviceIdType.MESH,
        )
        rdma.start()
        sends.append(rdma)

    for k in range(1, N_DEV):
        peer = (me + k) % N_DEV
        recv = pltpu.make_async_remote_copy(
            src_ref=p_ref,
            dst_ref=comm_ref.at[peer],
            send_sem=send_sems.at[k - 1],
            recv_sem=recv_sems.at[peer],
            device_id=(peer,),
            device_id_type=pl.DeviceIdType.MESH,
        )
        recv.wait_recv()
    for s in sends:
        s.wait_send()

    total = comm_ref[0] + comm_ref[1] + comm_ref[2] + comm_ref[3]
    out_ref[...] = lax.rsqrt(total * (1.0 / N_GLOBAL) + EPS)


def _norm_body(x_ref, gamma_ref, scale_ref, out_ref):
    out_ref[...] = x_ref[...] * gamma_ref[...] * scale_ref[...]


def kernel(x, gamma):
    m, n_local = x.shape
    n_blocks = m // BLOCK_M
    rows_c = m // 128

    part = pl.pallas_call(
        _partial_body,
        grid=(n_blocks,),
        in_specs=[pl.BlockSpec((BLOCK_M, n_local), lambda b: (b, 0))],
        out_specs=pl.BlockSpec((BLOCK_M, 1), lambda b: (b, 0)),
        out_shape=jax.ShapeDtypeStruct((m, 1), jnp.float32),
    )(x)

    part_c = part.reshape(rows_c, 128)

    scale_c = pl.pallas_call(
        _allreduce_body,
        out_shape=jax.ShapeDtypeStruct((rows_c, 128), jnp.float32),
        in_specs=[pl.BlockSpec(memory_space=pltpu.VMEM)],
        out_specs=pl.BlockSpec(memory_space=pltpu.VMEM),
        scratch_shapes=[
            pltpu.VMEM((N_DEV, rows_c, 128), jnp.float32),
            pltpu.SemaphoreType.DMA((N_DEV - 1,)),
            pltpu.SemaphoreType.DMA((N_DEV,)),
        ],
        compiler_params=pltpu.CompilerParams(collective_id=0),
    )(part_c)

    scale = scale_c.reshape(m, 1)
    gamma2 = gamma.reshape(1, n_local)

    return pl.pallas_call(
        _norm_body,
        grid=(n_blocks,),
        in_specs=[
            pl.BlockSpec((BLOCK_M, n_local), lambda b: (b, 0)),
            pl.BlockSpec((1, n_local), lambda b: (0, 0)),
            pl.BlockSpec((BLOCK_M, 1), lambda b: (b, 0)),
        ],
        out_specs=pl.BlockSpec((BLOCK_M, n_local), lambda b: (b, 0)),
        out_shape=jax.ShapeDtypeStruct((m, n_local), jnp.float32),
    )(x, gamma2, scale)
